# baseline (speedup 1.0000x reference)
import jax
import jax.numpy as jnp
from jax import lax
from jax.experimental import pallas as pl
from jax.experimental.pallas import tpu as pltpu
from jax.experimental.pallas import tpu_sc as plsc

NUM_FIELDS = 26
VOCAB = 100000
EMBED_DIM = 32
BATCH = 16384

NC, NS, L = 2, 16, 16
NW = NC * NS
B_PER_W = BATCH // NW
CB = 128
ROWS = CB * NUM_FIELDS
GIDX = 128
NG = ROWS // GIDX
NCH = B_PER_W // CB


def _body(idx_hbm, tab_hbm, out_hbm, idxs_v, perm_v, rows_v, sem_g):
    wid = lax.axis_index("s") * NC + lax.axis_index("c")
    base_b = wid * B_PER_W

    lane26 = lax.iota(jnp.int32, L) * NUM_FIELDS

    def chunk_body(c, carry):
        b0 = base_b + c * CB
        pltpu.sync_copy(idx_hbm.at[:, pl.ds(b0, CB)], idxs_v)
        for f in range(NUM_FIELDS):
            for g in range(CB // L):
                v = idxs_v[f, pl.ds(g * L, L)] + jnp.int32(f * VOCAB)
                pos = lane26 + jnp.int32(g * L * NUM_FIELDS + f)
                plsc.store_scatter(perm_v, [pos], v)
        copies = [
            pltpu.async_copy(
                tab_hbm.at[perm_v.at[pl.ds(j * GIDX, GIDX)]],
                rows_v.at[pl.ds(j * GIDX, GIDX)],
                sem_g,
            )
            for j in range(NG)
        ]
        for cp in copies:
            cp.wait()
        pltpu.sync_copy(
            rows_v, out_hbm.at[pl.ds((base_b + c * CB) * NUM_FIELDS, ROWS)]
        )
        return carry

    lax.fori_loop(0, NCH, chunk_body, 0)


def kernel(indices, tables):
    idx = indices.astype(jnp.int32)
    tab_flat = tables.reshape(NUM_FIELDS * VOCAB, EMBED_DIM)
    mesh = plsc.VectorSubcoreMesh(
        core_axis_name="c", subcore_axis_name="s", num_cores=NC, num_subcores=NS
    )
    out = pl.kernel(
        _body,
        out_type=jax.ShapeDtypeStruct((BATCH * NUM_FIELDS, EMBED_DIM), jnp.float32),
        mesh=mesh,
        compiler_params=pltpu.CompilerParams(
            needs_layout_passes=False, use_tc_tiling_on_sc=False
        ),
        scratch_types=[
            pltpu.VMEM((NUM_FIELDS, CB), jnp.int32),
            pltpu.VMEM((ROWS,), jnp.int32),
            pltpu.VMEM((ROWS, EMBED_DIM), jnp.float32),
            pltpu.SemaphoreType.DMA,
        ],
    )(idx, tab_flat)
    return out.reshape(BATCH, NUM_FIELDS, EMBED_DIM)


# speedup vs baseline: 1.0008x; 1.0008x over previous
import jax
import jax.numpy as jnp
from jax import lax
from jax.experimental import pallas as pl
from jax.experimental.pallas import tpu as pltpu
from jax.experimental.pallas import tpu_sc as plsc

NUM_FIELDS = 26
VOCAB = 100000
EMBED_DIM = 32
BATCH = 16384

NC, NS, L = 2, 16, 16
NW = NC * NS
B_PER_W = BATCH // NW
CB = 128
NCH = B_PER_W // CB


def _body(idx_hbm, tab_hbm, out_hbm, idxs_v, rows_v, sem_g, sem_o):
    wid = lax.axis_index("s") * NC + lax.axis_index("c")
    base_b = wid * B_PER_W

    def chunk_body(c, carry):
        b0 = base_b + c * CB
        pltpu.sync_copy(idx_hbm.at[:, pl.ds(b0, CB)], idxs_v)
        gathers = [
            pltpu.async_copy(
                tab_hbm.at[f].at[idxs_v.at[f]], rows_v.at[f], sem_g
            )
            for f in range(NUM_FIELDS)
        ]
        for cp in gathers:
            cp.wait()
        stores = [
            pltpu.async_copy(
                rows_v.at[f], out_hbm.at[pl.ds(b0, CB), f], sem_o
            )
            for f in range(NUM_FIELDS)
        ]
        for cp in stores:
            cp.wait()
        return carry

    lax.fori_loop(0, NCH, chunk_body, 0)


def kernel(indices, tables):
    idx = indices.astype(jnp.int32)
    mesh = plsc.VectorSubcoreMesh(
        core_axis_name="c", subcore_axis_name="s", num_cores=NC, num_subcores=NS
    )
    out = pl.kernel(
        _body,
        out_type=jax.ShapeDtypeStruct((BATCH, NUM_FIELDS, EMBED_DIM), jnp.float32),
        mesh=mesh,
        compiler_params=pltpu.CompilerParams(
            needs_layout_passes=False, use_tc_tiling_on_sc=False
        ),
        scratch_types=[
            pltpu.VMEM((NUM_FIELDS, CB), jnp.int32),
            pltpu.VMEM((NUM_FIELDS, CB, EMBED_DIM), jnp.float32),
            pltpu.SemaphoreType.DMA,
            pltpu.SemaphoreType.DMA,
        ],
    )(idx, tables)
    return out
